# Initial kernel scaffold; baseline (speedup 1.0000x reference)
import jax
import jax.numpy as jnp
from jax.experimental import pallas as pl


def kernel(feature, edge_weight, W, b, edge_index):
    raise NotImplementedError("write your pallas kernel here")


# speedup vs baseline: 6.0175x; 6.0175x over previous
import functools

import jax
import jax.numpy as jnp
from jax import lax
from jax.experimental import pallas as pl
from jax.experimental.pallas import tpu as pltpu
from jax.experimental.pallas import tpu_sc as plsc

NC = 2
NS = 16
NW = NC * NS
C = 80
SC = 5


def _bcast_lane(v16, j):
    return lax.gather(
        v16,
        jnp.full((16, 1), j, jnp.int32),
        lax.GatherDimensionNumbers(
            offset_dims=(), collapsed_slice_dims=(0,), start_index_map=(0,)),
        slice_sizes=(1,),
        mode=lax.GatherScatterMode.PROMISE_IN_BOUNDS,
    )


def _make_spmm(N, D, E):
    EPW = E // NW
    NSUP = EPW // (SC * C)
    RPT = ((N + NS - 1) // NS + 7) // 8 * 8
    NP = RPT * NS
    ZR = RPT // 8
    LG = D // 16

    mesh = plsc.VectorSubcoreMesh(core_axis_name="c", subcore_axis_name="s")

    @functools.partial(
        pl.kernel,
        out_type=(
            jax.ShapeDtypeStruct((NP, D), jnp.float32),
            jax.ShapeDtypeStruct((NP, D), jnp.float32),
        ),
        mesh=mesh,
        scratch_types=[
            pltpu.VMEM((SC, C), jnp.int32),
            pltpu.VMEM((SC, C), jnp.int32),
            pltpu.VMEM((SC, C), jnp.float32),
            pltpu.VMEM((C, D), jnp.float32),
            pltpu.VMEM((ZR, D), jnp.float32),
            pltpu.VMEM_SHARED((NP, D), jnp.float32),
            pltpu.SemaphoreType.DMA,
        ],
    )
    def spmm(feat_hbm, src_hbm, dst_hbm, ew_hbm, out0, out1,
             src_v, dst_v, ew_v, rows_v, zbuf, acc, sem):
        c = lax.axis_index("c")
        s = lax.axis_index("s")
        wid = s * NC + c

        zeros16 = jnp.zeros((16,), jnp.float32)

        @pl.loop(0, ZR)
        def _(r):
            for g in range(LG):
                zbuf[r, pl.ds(g * 16, 16)] = zeros16

        for j in range(RPT // ZR):
            pltpu.sync_copy(zbuf, acc.at[pl.ds(s * RPT + j * ZR, ZR)])

        plsc.subcore_barrier()

        @pl.loop(0, NSUP)
        def _(sup):
            pltpu.sync_copy(src_hbm.at[wid, sup], src_v)
            pltpu.sync_copy(dst_hbm.at[wid, sup], dst_v)
            pltpu.sync_copy(ew_hbm.at[wid, sup], ew_v)

            for j in range(SC):
                pltpu.async_copy(feat_hbm.at[src_v.at[j]], rows_v, sem).wait()

                @pl.loop(0, C // 16)
                def _(e16):
                    wgrp = ew_v[j, pl.ds(e16 * 16, 16)]
                    for i in range(16):
                        wb = _bcast_lane(wgrp, i)
                        e = e16 * 16 + i
                        for g in range(LG):
                            sl = pl.ds(g * 16, 16)
                            rows_v[e, sl] = rows_v[e, sl] * wb

                pltpu.sync_copy(rows_v, acc.at[dst_v.at[j]], add=True)

        plsc.subcore_barrier()

        @pl.when(c == 0)
        def _():
            pltpu.sync_copy(acc.at[pl.ds(s * RPT, RPT)],
                            out0.at[pl.ds(s * RPT, RPT)])

        @pl.when(c == 1)
        def _():
            pltpu.sync_copy(acc.at[pl.ds(s * RPT, RPT)],
                            out1.at[pl.ds(s * RPT, RPT)])

    return spmm


def _combine_matmul_body(p0_ref, p1_ref, w_ref, b_ref, o_ref):
    x = p0_ref[...] + p1_ref[...]
    o_ref[...] = (
        jnp.dot(x, w_ref[...], preferred_element_type=jnp.float32)
        + b_ref[...]
    )


def _make_combine(N, D, BM):
    return pl.pallas_call(
        _combine_matmul_body,
        grid=(N // BM,),
        in_specs=[
            pl.BlockSpec((BM, D), lambda i: (i, 0)),
            pl.BlockSpec((BM, D), lambda i: (i, 0)),
            pl.BlockSpec((D, D), lambda i: (0, 0)),
            pl.BlockSpec((1, D), lambda i: (0, 0)),
        ],
        out_specs=pl.BlockSpec((BM, D), lambda i: (i, 0)),
        out_shape=jax.ShapeDtypeStruct((N, D), jnp.float32),
    )


@jax.jit
def kernel(feature, edge_weight, W, b, edge_index):
    N, D = feature.shape
    E = edge_weight.shape[0]
    EPW = E // NW
    NSUP = EPW // (SC * C)

    src = edge_index[1].reshape(NW, NSUP, SC, C)
    dst = edge_index[0].reshape(NW, NSUP, SC, C)
    ew = edge_weight.reshape(NW, NSUP, SC, C)

    p0, p1 = _make_spmm(N, D, E)(feature, src, dst, ew)
    return _make_combine(N, D, 1000)(p0, p1, W, b.reshape(1, D))


# speedup vs baseline: 10.7805x; 1.7915x over previous
import functools

import jax
import jax.numpy as jnp
from jax import lax
from jax.experimental import pallas as pl
from jax.experimental.pallas import tpu as pltpu
from jax.experimental.pallas import tpu_sc as plsc

NC = 2
NS = 16
NW = NC * NS
C = 80


def _bcast_lane(v16, j):
    return lax.gather(
        v16,
        jnp.full((16, 1), j, jnp.int32),
        lax.GatherDimensionNumbers(
            offset_dims=(), collapsed_slice_dims=(0,), start_index_map=(0,)),
        slice_sizes=(1,),
        mode=lax.GatherScatterMode.PROMISE_IN_BOUNDS,
    )


def _make_spmm(N, D, E):
    EPW = E // NW
    NITER = EPW // C
    NPH = 5
    CPP = NITER // NPH
    NPAIR = (CPP - 1) // 2
    RPT = ((N + NS - 1) // NS + 7) // 8 * 8
    NP = RPT * NS
    LG = D // 16

    mesh = plsc.VectorSubcoreMesh(core_axis_name="c", subcore_axis_name="s")

    @functools.partial(
        pl.kernel,
        out_type=(
            jax.ShapeDtypeStruct((NP, D), jnp.float32),
            jax.ShapeDtypeStruct((NP, D), jnp.float32),
        ),
        mesh=mesh,
        scratch_types=[
            pltpu.VMEM((2, CPP, C), jnp.int32),
            pltpu.VMEM((2, CPP, C), jnp.int32),
            pltpu.VMEM((2, CPP, C), jnp.float32),
            pltpu.VMEM((C, D), jnp.float32),
            pltpu.VMEM((C, D), jnp.float32),
            pltpu.SemaphoreType.DMA,
            pltpu.SemaphoreType.DMA,
            pltpu.SemaphoreType.DMA,
            pltpu.SemaphoreType.DMA,
            pltpu.VMEM_SHARED((NP, D), jnp.float32),
        ],
    )
    def spmm(feat_hbm, src_hbm, dst_hbm, ew_hbm, out0, out1,
             src_v, dst_v, ew_v, rows0, rows1, sem0, sem1, ssemA, ssemB, acc):
        c = lax.axis_index("c")
        s = lax.axis_index("s")
        wid = s * NC + c

        bufs = [(rows0, sem0), (rows1, sem1)]
        ssems = [ssemA, ssemB]
        zeros16 = jnp.zeros((16,), jnp.float32)

        @pl.loop(0, C)
        def _(r):
            for g in range(LG):
                rows0[r, pl.ds(g * 16, 16)] = zeros16

        for j in range(RPT // C):
            pltpu.sync_copy(rows0, acc.at[pl.ds(s * RPT + j * C, C)])
        rem = RPT % C
        if rem:
            pltpu.sync_copy(rows0.at[pl.ds(0, rem)],
                            acc.at[pl.ds(s * RPT + (RPT // C) * C, rem)])

        def stage_copies(ph, sp):
            return [
                pltpu.make_async_copy(src_hbm.at[wid, ph], src_v.at[sp],
                                      ssems[sp]),
                pltpu.make_async_copy(dst_hbm.at[wid, ph], dst_v.at[sp],
                                      ssems[sp]),
                pltpu.make_async_copy(ew_hbm.at[wid, ph], ew_v.at[sp],
                                      ssems[sp]),
            ]

        pltpu.sync_copy(src_hbm.at[wid, 0], src_v.at[0])
        pltpu.sync_copy(dst_hbm.at[wid, 0], dst_v.at[0])
        pltpu.sync_copy(ew_hbm.at[wid, 0], ew_v.at[0])

        plsc.subcore_barrier()

        def scale(buf, sp, l):
            @pl.loop(0, C // 16)
            def _(e16):
                wgrp = ew_v[sp, l, pl.ds(e16 * 16, 16)]
                for i in range(16):
                    wb = _bcast_lane(wgrp, i)
                    e = e16 * 16 + i
                    for q in range(LG):
                        sl = pl.ds(q * 16, 16)
                        buf[e, sl] = buf[e, sl] * wb

        def fire(bp, sp, l):
            buf, sem = bufs[bp]
            pltpu.async_copy(feat_hbm.at[src_v.at[sp, l]], buf, sem)

        def wait(bp, sp, l):
            buf, sem = bufs[bp]
            pltpu.make_async_copy(feat_hbm.at[src_v.at[sp, l]], buf,
                                  sem).wait()

        def process(bp, sp, l):
            buf, _ = bufs[bp]
            wait(bp, sp, l)
            scale(buf, sp, l)
            pltpu.sync_copy(buf, acc.at[dst_v.at[sp, l]], add=True)

        fire(0, 0, 0)

        par = 0
        for ph in range(NPH):
            sp = ph % 2
            spn = (ph + 1) % 2
            if ph + 1 < NPH:
                for cp in stage_copies(ph + 1, spn):
                    cp.start()

            @pl.loop(0, NPAIR)
            def _(p, par=par, sp=sp):
                l0 = 2 * p
                fire(par ^ 1, sp, l0 + 1)
                process(par, sp, l0)
                fire(par, sp, l0 + 2)
                process(par ^ 1, sp, l0 + 1)

            if ph + 1 < NPH:
                for cp in stage_copies(ph + 1, spn):
                    cp.wait()
                fire(par ^ 1, spn, 0)
            process(par, sp, CPP - 1)
            par ^= 1

        plsc.subcore_barrier()

        @pl.when(c == 0)
        def _():
            pltpu.sync_copy(acc.at[pl.ds(s * RPT, RPT)],
                            out0.at[pl.ds(s * RPT, RPT)])

        @pl.when(c == 1)
        def _():
            pltpu.sync_copy(acc.at[pl.ds(s * RPT, RPT)],
                            out1.at[pl.ds(s * RPT, RPT)])

    return spmm


def _combine_matmul_body(p0_ref, p1_ref, w_ref, b_ref, o_ref):
    x = p0_ref[...] + p1_ref[...]
    o_ref[...] = (
        jnp.dot(x, w_ref[...], preferred_element_type=jnp.float32)
        + b_ref[...]
    )


def _make_combine(N, D, BM):
    return pl.pallas_call(
        _combine_matmul_body,
        grid=(N // BM,),
        in_specs=[
            pl.BlockSpec((BM, D), lambda i: (i, 0)),
            pl.BlockSpec((BM, D), lambda i: (i, 0)),
            pl.BlockSpec((D, D), lambda i: (0, 0)),
            pl.BlockSpec((1, D), lambda i: (0, 0)),
        ],
        out_specs=pl.BlockSpec((BM, D), lambda i: (i, 0)),
        out_shape=jax.ShapeDtypeStruct((N, D), jnp.float32),
    )


@jax.jit
def kernel(feature, edge_weight, W, b, edge_index):
    N, D = feature.shape
    E = edge_weight.shape[0]
    EPW = E // NW
    NITER = EPW // C

    NPH = 5
    CPP = NITER // NPH
    src = edge_index[1].reshape(NW, NPH, CPP, C)
    dst = edge_index[0].reshape(NW, NPH, CPP, C)
    ew = edge_weight.reshape(NW, NPH, CPP, C)

    p0, p1 = _make_spmm(N, D, E)(feature, src, dst, ew)
    return _make_combine(N, D, 1000)(p0, p1, W, b.reshape(1, D))
